# Initial kernel scaffold; baseline (speedup 1.0000x reference)
import functools

import jax
import jax.numpy as jnp
from jax import lax
from jax.experimental import pallas as pl
from jax.experimental.pallas import tpu as pltpu
from jax.experimental.pallas import tpu_sc as plsc

D = 100000
N = 128
B = 16384
ARITY = 3

NC = 2
NS = 16
NW = NC * NS
QPW = B // NW
CH = 128
NCH = QPW // CH
LANES = 16


@functools.partial(
    pl.kernel,
    mesh=plsc.VectorSubcoreMesh(core_axis_name="c", subcore_axis_name="s"),
    out_type=jax.ShapeDtypeStruct((B,), jnp.float32),
    scratch_types=[
        pltpu.VMEM((ARITY, NCH, CH), jnp.int32),
        pltpu.VMEM((ARITY, CH, N), jnp.float32),
        pltpu.VMEM((QPW,), jnp.float32),
        pltpu.SemaphoreType.DMA,
    ],
)
def _rap_sc(idx_hbm, w_hbm, out_hbm, idx_v, rows_v, out_v, sem):
    wid = lax.axis_index("s") * NC + lax.axis_index("c")
    base = wid * QPW

    pltpu.sync_copy(idx_hbm.at[wid], idx_v)

    for c in range(NCH):
        copies = [
            pltpu.async_copy(w_hbm.at[idx_v.at[a, c]], rows_v.at[a], sem)
            for a in range(ARITY)
        ]
        for cp in copies:
            cp.wait()

        def qbody(q, _, c=c):
            acc = jnp.zeros((LANES,), jnp.float32)
            for j in range(N // LANES):
                sl = pl.ds(j * LANES, LANES)
                acc = acc + (rows_v[0, q, sl] * rows_v[1, q, sl]
                             * rows_v[2, q, sl])
            out_v[c * CH + q] = jnp.sum(acc) * (1.0 / N)
            return 0

        lax.fori_loop(0, CH, qbody, 0)

    pltpu.sync_copy(out_v, out_hbm.at[pl.ds(base, QPW)])


def kernel(q_t_idxs, W):
    idx = q_t_idxs.astype(jnp.int32)
    idx = idx.reshape(NW, NCH, CH, ARITY).transpose(0, 3, 1, 2)
    return _rap_sc(idx, W)


# speedup vs baseline: 2.9599x; 2.9599x over previous
import functools

import jax
import jax.numpy as jnp
from jax import lax
from jax.experimental import pallas as pl
from jax.experimental.pallas import tpu as pltpu
from jax.experimental.pallas import tpu_sc as plsc

D = 100000
N = 128
B = 16384
ARITY = 3

NC = 2
NS = 16
NW = NC * NS
QPW = B // NW
CH = 128
NCH = QPW // CH
LANES = 16


@functools.partial(
    pl.kernel,
    mesh=plsc.VectorSubcoreMesh(core_axis_name="c", subcore_axis_name="s"),
    out_type=jax.ShapeDtypeStruct((B, LANES), jnp.float32),
    scratch_types=[
        pltpu.VMEM((ARITY, NCH, CH), jnp.int32),
        pltpu.VMEM((CH, N), jnp.float32),
        pltpu.VMEM((CH, N), jnp.float32),
        pltpu.VMEM((CH, N), jnp.float32),
        pltpu.VMEM((QPW, LANES), jnp.float32),
        pltpu.SemaphoreType.DMA,
    ],
)
def _rap_sc(idx_hbm, w_hbm, out_hbm, idx_v, r0, r1, r2, out_v, sem):
    wid = lax.axis_index("s") * NC + lax.axis_index("c")
    base = wid * QPW

    pltpu.sync_copy(idx_hbm.at[wid], idx_v)

    for c in range(NCH):
        copies = [
            pltpu.async_copy(w_hbm.at[idx_v.at[a, c]], r, sem)
            for a, r in ((0, r0), (1, r1), (2, r2))
        ]
        for cp in copies:
            cp.wait()

        def qbody(q, _, c=c):
            acc = jnp.zeros((LANES,), jnp.float32)
            for j in range(N // LANES):
                sl = pl.ds(j * LANES, LANES)
                acc = acc + r0[q, sl] * r1[q, sl] * r2[q, sl]
            out_v[c * CH + q, :] = acc
            return 0

        lax.fori_loop(0, CH, qbody, 0)

    pltpu.sync_copy(out_v, out_hbm.at[pl.ds(base, QPW)])


def _tc_body(p_ref, o_ref):
    ones = jnp.full((LANES, 1), 1.0 / N, jnp.float32)
    o_ref[...] = jax.lax.dot_general(
        p_ref[...], ones, (((1,), (0,)), ((), ())),
        preferred_element_type=jnp.float32)


def _tc_reduce(partials):
    out = pl.pallas_call(
        _tc_body,
        out_shape=jax.ShapeDtypeStruct((B, 1), jnp.float32),
    )(partials)
    return out.reshape(B)


def kernel(q_t_idxs, W):
    idx = q_t_idxs.astype(jnp.int32)
    idx = idx.reshape(NW, NCH, CH, ARITY).transpose(0, 3, 1, 2)
    partials = _rap_sc(idx, W)
    return _tc_reduce(partials)


# speedup vs baseline: 3.2171x; 1.0869x over previous
import functools

import jax
import jax.numpy as jnp
from jax import lax
from jax.experimental import pallas as pl
from jax.experimental.pallas import tpu as pltpu
from jax.experimental.pallas import tpu_sc as plsc

D = 100000
N = 128
B = 16384
ARITY = 3

NC = 2
NS = 16
NW = NC * NS
QPW = B // NW
CH = 64
NCH = QPW // CH
LANES = 16


@functools.partial(
    pl.kernel,
    mesh=plsc.VectorSubcoreMesh(core_axis_name="c", subcore_axis_name="s"),
    out_type=jax.ShapeDtypeStruct((B, LANES), jnp.float32),
    scratch_types=[
        pltpu.VMEM((ARITY, NCH, CH), jnp.int32),
        pltpu.VMEM((CH, N), jnp.float32),
        pltpu.VMEM((CH, N), jnp.float32),
        pltpu.VMEM((CH, N), jnp.float32),
        pltpu.VMEM((CH, N), jnp.float32),
        pltpu.VMEM((CH, N), jnp.float32),
        pltpu.VMEM((CH, N), jnp.float32),
        pltpu.VMEM((QPW, LANES), jnp.float32),
        pltpu.SemaphoreType.DMA,
        pltpu.SemaphoreType.DMA,
    ],
)
def _rap_sc(idx_hbm, w_hbm, out_hbm, idx_v,
            s0a, s0b, s0c, s1a, s1b, s1c, out_v, sem0, sem1):
    wid = lax.axis_index("s") * NC + lax.axis_index("c")
    base = wid * QPW
    slots = ((s0a, s0b, s0c), (s1a, s1b, s1c))
    sems = (sem0, sem1)

    pltpu.sync_copy(idx_hbm.at[wid], idx_v)

    def fire(c):
        return [
            pltpu.async_copy(w_hbm.at[idx_v.at[a, c]], slots[c % 2][a],
                             sems[c % 2])
            for a in range(ARITY)
        ]

    pending = {0: fire(0)}
    for c in range(NCH):
        if c + 1 < NCH:
            pending[c + 1] = fire(c + 1)
        for cp in pending.pop(c):
            cp.wait()

        r0, r1, r2 = slots[c % 2]

        def qbody(q, _, c=c, r0=r0, r1=r1, r2=r2):
            acc = jnp.zeros((LANES,), jnp.float32)
            for j in range(N // LANES):
                sl = pl.ds(j * LANES, LANES)
                acc = acc + r0[q, sl] * r1[q, sl] * r2[q, sl]
            out_v[c * CH + q, :] = acc
            return 0

        lax.fori_loop(0, CH, qbody, 0)

    pltpu.sync_copy(out_v, out_hbm.at[pl.ds(base, QPW)])


def _tc_body(p_ref, o_ref):
    ones = jnp.full((LANES, 1), 1.0 / N, jnp.float32)
    o_ref[...] = jax.lax.dot_general(
        p_ref[...], ones, (((1,), (0,)), ((), ())),
        preferred_element_type=jnp.float32)


def _tc_reduce(partials):
    out = pl.pallas_call(
        _tc_body,
        out_shape=jax.ShapeDtypeStruct((B, 1), jnp.float32),
    )(partials)
    return out.reshape(B)


def kernel(q_t_idxs, W):
    idx = q_t_idxs.astype(jnp.int32)
    idx = idx.reshape(NW, NCH, CH, ARITY).transpose(0, 3, 1, 2)
    partials = _rap_sc(idx, W)
    return _tc_reduce(partials)


# speedup vs baseline: 3.8069x; 1.1833x over previous
import functools

import jax
import jax.numpy as jnp
from jax import lax
from jax.experimental import pallas as pl
from jax.experimental.pallas import tpu as pltpu
from jax.experimental.pallas import tpu_sc as plsc

D = 100000
N = 128
B = 16384
ARITY = 3

NC = 2
NS = 16
NW = NC * NS
QPW = B // NW
CH = 64
NCH = QPW // CH
LANES = 16


@functools.partial(
    pl.kernel,
    mesh=plsc.VectorSubcoreMesh(core_axis_name="c", subcore_axis_name="s"),
    out_type=jax.ShapeDtypeStruct((B, LANES), jnp.float32),
    scratch_types=[
        pltpu.VMEM((ARITY, NCH, CH), jnp.int32),
        pltpu.VMEM((CH, N), jnp.float32),
        pltpu.VMEM((CH, N), jnp.float32),
        pltpu.VMEM((CH, N), jnp.float32),
        pltpu.VMEM((CH, N), jnp.float32),
        pltpu.VMEM((CH, N), jnp.float32),
        pltpu.VMEM((CH, N), jnp.float32),
        pltpu.VMEM((QPW, LANES), jnp.float32),
        pltpu.SemaphoreType.DMA,
        pltpu.SemaphoreType.DMA,
    ],
)
def _rap_sc(idx_hbm, w_hbm, out_hbm, idx_v,
            s0a, s0b, s0c, s1a, s1b, s1c, out_v, sem0, sem1):
    wid = lax.axis_index("s") * NC + lax.axis_index("c")
    base = wid * QPW
    slots = ((s0a, s0b, s0c), (s1a, s1b, s1c))
    sems = (sem0, sem1)

    pltpu.sync_copy(idx_hbm.at[wid], idx_v)

    def fire(c):
        return [
            pltpu.async_copy(w_hbm.at[idx_v.at[a, c]], slots[c % 2][a],
                             sems[c % 2])
            for a in range(ARITY)
        ]

    pending = {0: fire(0)}
    for c in range(NCH):
        if c + 1 < NCH:
            pending[c + 1] = fire(c + 1)
        for cp in pending.pop(c):
            cp.wait()

        r0, r1, r2 = slots[c % 2]

        def qbody(q, _, c=c, r0=r0, r1=r1, r2=r2):
            acc = jnp.zeros((LANES,), jnp.float32)
            for j in range(N // LANES):
                sl = pl.ds(j * LANES, LANES)
                acc = acc + r0[q, sl] * r1[q, sl] * r2[q, sl]
            out_v[c * CH + q, :] = acc
            return 0

        lax.fori_loop(0, CH, qbody, 0)

    pltpu.sync_copy(out_v, out_hbm.at[pl.ds(base, QPW)])


def _tc_body(p_ref, o_ref):
    ones = jnp.full((1, LANES), 1.0 / N, jnp.float32)
    o_ref[...] = jax.lax.dot_general(
        ones, p_ref[...], (((1,), (1,)), ((), ())),
        preferred_element_type=jnp.float32)


def _tc_reduce(partials):
    out = pl.pallas_call(
        _tc_body,
        out_shape=jax.ShapeDtypeStruct((1, B), jnp.float32),
    )(partials)
    return out.reshape(B)


def kernel(q_t_idxs, W):
    idx = q_t_idxs.astype(jnp.int32)
    idx = idx.reshape(NW, NCH, CH, ARITY).transpose(0, 3, 1, 2)
    partials = _rap_sc(idx, W)
    return _tc_reduce(partials)
